# Initial kernel scaffold; baseline (speedup 1.0000x reference)
import jax
import jax.numpy as jnp
from jax.experimental import pallas as pl


def kernel(x_joint, x_torso, edge_index_tj, edge_index_jt, edge_index_jj, Wj, bj, Wt, bt, Wrel_tj, brel_tj, Wroot_tj, Wrel_jt, brel_jt, Wroot_jt, Wrel_jj, brel_jj, Wroot_jj, Wout, bout):
    raise NotImplementedError("write your pallas kernel here")


# speedup vs baseline: 15.8265x; 15.8265x over previous
import functools

import numpy as np
import jax
import jax.numpy as jnp
from jax import lax
from jax.experimental import pallas as pl
from jax.experimental.pallas import tpu as pltpu
from jax.experimental.pallas import tpu_sc as plsc

N_J = 80000
N_T = 20000
_PAD = 128
NJP = N_J + _PAD
NTP = N_T + _PAD
NC = 2
NS = 16
NW = NC * NS
CH = 4096
TJ_CH = 5
JJ_CH = 15
E_TJ_P = NW * TJ_CH * CH
E_JJ_P = NW * JJ_CH * CH
RPT = NJP // NS
_SP_BIAS = float(np.log(np.exp(1.0) - 1.0))


def _pre_joint_body(woutT_ref, wreljjT_ref, wrtjT_ref, wrjjT_ref, wjT_ref,
                    bj_ref, brel_ref, bout_ref, x_ref, o_ref):
    ajjT = jnp.dot(woutT_ref[...], wreljjT_ref[...],
                   preferred_element_type=jnp.float32)
    arT = jnp.dot(woutT_ref[...], wrtjT_ref[...] + wrjjT_ref[...],
                  preferred_element_type=jnp.float32)
    gjT = jnp.dot(ajjT, wjT_ref[...], preferred_element_type=jnp.float32)
    grT = jnp.dot(arT, wjT_ref[...], preferred_element_type=jnp.float32)
    cj = jnp.dot(ajjT, bj_ref[...], preferred_element_type=jnp.float32)
    cr = (jnp.dot(arT, bj_ref[...], preferred_element_type=jnp.float32)
          + jnp.dot(woutT_ref[...], brel_ref[...],
                    preferred_element_type=jnp.float32)
          + bout_ref[...])
    g4 = jnp.concatenate([gjT, grT], axis=0)
    c4 = jnp.concatenate([cj, cr], axis=0)
    x = x_ref[...]
    o_ref[...] = jnp.dot(g4, x, preferred_element_type=jnp.float32) + c4


def _pre_torso_body(woutT_ref, wreltjT_ref, wtT_ref, bt_ref, x_ref, o_ref):
    atjT = jnp.dot(woutT_ref[...], wreltjT_ref[...],
                   preferred_element_type=jnp.float32)
    gtT = jnp.dot(atjT, wtT_ref[...], preferred_element_type=jnp.float32)
    ct = jnp.dot(atjT, bt_ref[...], preferred_element_type=jnp.float32)
    x = x_ref[...]
    o_ref[...] = jnp.dot(gtT, x, preferred_element_type=jnp.float32) + ct


def _post_body(p0_ref, p1_ref, loc_ref, scale_ref):
    loc_ref[...] = p0_ref[0:1, :] + p0_ref[1:2, :]
    s = p1_ref[0:1, :] + p1_ref[1:2, :] + _SP_BIAS
    scale_ref[...] = jax.nn.softplus(s)


def _sc_body(yt0, yt1, yj0, yj1, stj, dtj, sjj, djj, init0, init1,
             out0, out1, src_v, dst_v, g0_v, g1_v, stage_v, acc0, acc1, sem):
    c = lax.axis_index("c")
    s = lax.axis_index("s")
    wid = s * NC + c
    base = s * RPT
    hb = c * NJP + base
    pltpu.sync_copy(init0.at[pl.ds(hb, RPT)], stage_v)
    pltpu.sync_copy(stage_v, acc0.at[pl.ds(base, RPT)])
    pltpu.sync_copy(init1.at[pl.ds(hb, RPT)], stage_v)
    pltpu.sync_copy(stage_v, acc1.at[pl.ds(base, RPT)])
    plsc.subcore_barrier()

    def do_edges(src_h, dst_h, t0, t1, nchunks):
        for i in range(nchunks):
            off = (wid * nchunks + i) * CH
            pltpu.sync_copy(src_h.at[pl.ds(off, CH)], src_v)
            pltpu.sync_copy(dst_h.at[pl.ds(off, CH)], dst_v)
            pltpu.async_copy(t0.at[src_v], g0_v, sem).wait()
            pltpu.async_copy(t1.at[src_v], g1_v, sem).wait()
            pltpu.sync_copy(g0_v, acc0.at[dst_v], add=True)
            pltpu.sync_copy(g1_v, acc1.at[dst_v], add=True)

    do_edges(stj, dtj, yt0, yt1, TJ_CH)
    do_edges(sjj, djj, yj0, yj1, JJ_CH)
    plsc.subcore_barrier()
    pltpu.sync_copy(acc0.at[pl.ds(base, RPT)], stage_v)
    pltpu.sync_copy(stage_v, out0.at[pl.ds(hb, RPT)])
    pltpu.sync_copy(acc1.at[pl.ds(base, RPT)], stage_v)
    pltpu.sync_copy(stage_v, out1.at[pl.ds(hb, RPT)])


_sc_scatter = functools.partial(
    pl.kernel,
    mesh=plsc.VectorSubcoreMesh(core_axis_name="c", subcore_axis_name="s"),
    out_type=[jax.ShapeDtypeStruct((NC * NJP,), jnp.float32),
              jax.ShapeDtypeStruct((NC * NJP,), jnp.float32)],
    scratch_types=[
        pltpu.VMEM((CH,), jnp.int32),
        pltpu.VMEM((CH,), jnp.int32),
        pltpu.VMEM((CH,), jnp.float32),
        pltpu.VMEM((CH,), jnp.float32),
        pltpu.VMEM((RPT,), jnp.float32),
        pltpu.VMEM_SHARED((NJP,), jnp.float32),
        pltpu.VMEM_SHARED((NJP,), jnp.float32),
        pltpu.SemaphoreType.DMA,
    ],
)(_sc_body)


def _pad_edges(ei, e_pad, n_src, n_dst):
    e = ei.shape[1]
    ar = jnp.arange(e_pad - e, dtype=jnp.int32)
    src = jnp.concatenate([ei[0].astype(jnp.int32), n_src + ar % _PAD])
    dst = jnp.concatenate([ei[1].astype(jnp.int32), n_dst + ar % _PAD])
    return src, dst


def kernel(x_joint, x_torso, edge_index_tj, edge_index_jt, edge_index_jj,
           Wj, bj, Wt, bt, Wrel_tj, brel_tj, Wroot_tj,
           Wrel_jt, brel_jt, Wroot_jt, Wrel_jj, brel_jj, Wroot_jj,
           Wout, bout):
    f32 = jnp.float32
    xjT = x_joint.T
    xtT = x_torso.T
    woutT = Wout.T
    brel_col = (brel_tj + brel_jj).reshape(64, 1)
    bj_col = bj.reshape(11, 1)
    bt_col = bt.reshape(11, 1)
    bout_col = bout.reshape(2, 1)

    blkj = 16000
    prej = pl.pallas_call(
        _pre_joint_body,
        grid=(N_J // blkj,),
        in_specs=[pl.BlockSpec((2, 64), lambda i: (0, 0)),
                  pl.BlockSpec((64, 11), lambda i: (0, 0)),
                  pl.BlockSpec((64, 11), lambda i: (0, 0)),
                  pl.BlockSpec((64, 11), lambda i: (0, 0)),
                  pl.BlockSpec((11, 2), lambda i: (0, 0)),
                  pl.BlockSpec((11, 1), lambda i: (0, 0)),
                  pl.BlockSpec((64, 1), lambda i: (0, 0)),
                  pl.BlockSpec((2, 1), lambda i: (0, 0)),
                  pl.BlockSpec((2, blkj), lambda i: (0, i))],
        out_specs=pl.BlockSpec((4, blkj), lambda i: (0, i)),
        out_shape=jax.ShapeDtypeStruct((4, N_J), f32),
    )(woutT, Wrel_jj.T, Wroot_tj.T, Wroot_jj.T, Wj.T,
      bj_col, brel_col, bout_col, xjT)

    pret = pl.pallas_call(
        _pre_torso_body,
        grid=(1,),
        in_specs=[pl.BlockSpec((2, 64), lambda i: (0, 0)),
                  pl.BlockSpec((64, 11), lambda i: (0, 0)),
                  pl.BlockSpec((11, 11), lambda i: (0, 0)),
                  pl.BlockSpec((11, 1), lambda i: (0, 0)),
                  pl.BlockSpec((11, N_T), lambda i: (0, 0))],
        out_specs=pl.BlockSpec((2, N_T), lambda i: (0, 0)),
        out_shape=jax.ShapeDtypeStruct((2, N_T), f32),
    )(woutT, Wrel_tj.T, Wt.T, bt_col, xtT)

    zpadj = jnp.zeros((_PAD,), f32)
    zrow = jnp.zeros((NJP,), f32)
    yj0 = jnp.concatenate([prej[0], zpadj])
    yj1 = jnp.concatenate([prej[1], zpadj])
    init0 = jnp.concatenate([prej[2], zpadj, zrow])
    init1 = jnp.concatenate([prej[3], zpadj, zrow])
    yt0 = jnp.concatenate([pret[0], zpadj])
    yt1 = jnp.concatenate([pret[1], zpadj])

    stj, dtj = _pad_edges(edge_index_tj, E_TJ_P, N_T, N_J)
    sjj, djj = _pad_edges(edge_index_jj, E_JJ_P, N_J, N_J)

    p0, p1 = _sc_scatter(yt0, yt1, yj0, yj1, stj, dtj, sjj, djj, init0, init1)
    p0 = p0.reshape(NC, NJP)
    p1 = p1.reshape(NC, NJP)

    blk = 16000
    loc2, scale2 = pl.pallas_call(
        _post_body,
        grid=(N_J // blk,),
        in_specs=[pl.BlockSpec((2, blk), lambda i: (0, i)),
                  pl.BlockSpec((2, blk), lambda i: (0, i))],
        out_specs=[pl.BlockSpec((1, blk), lambda i: (0, i)),
                   pl.BlockSpec((1, blk), lambda i: (0, i))],
        out_shape=[jax.ShapeDtypeStruct((1, N_J), f32),
                   jax.ShapeDtypeStruct((1, N_J), f32)],
    )(p0, p1)
    return (loc2.reshape(N_J), scale2.reshape(N_J))


# speedup vs baseline: 36.1675x; 2.2852x over previous
import functools

import numpy as np
import jax
import jax.numpy as jnp
from jax import lax
from jax.experimental import pallas as pl
from jax.experimental.pallas import tpu as pltpu
from jax.experimental.pallas import tpu_sc as plsc

N_J = 80000
N_T = 20000
_PAD = 2048
NJP = N_J + _PAD
NTP = N_T + _PAD
NC = 2
NS = 16
NW = NC * NS
CH = 10240
TJ_CH = 2
JJ_CH = 6
E_TJ_P = NW * TJ_CH * CH
E_JJ_P = NW * JJ_CH * CH
RPT = NJP // NS
_SP_BIAS = float(np.log(np.exp(1.0) - 1.0))


def _pre_joint_body(woutT_ref, wreljjT_ref, wrtjT_ref, wrjjT_ref, wjT_ref,
                    bj_ref, brel_ref, bout_ref, x_ref, o_ref):
    ajjT = jnp.dot(woutT_ref[...], wreljjT_ref[...],
                   preferred_element_type=jnp.float32)
    arT = jnp.dot(woutT_ref[...], wrtjT_ref[...] + wrjjT_ref[...],
                  preferred_element_type=jnp.float32)
    gjT = jnp.dot(ajjT, wjT_ref[...], preferred_element_type=jnp.float32)
    grT = jnp.dot(arT, wjT_ref[...], preferred_element_type=jnp.float32)
    cj = jnp.dot(ajjT, bj_ref[...], preferred_element_type=jnp.float32)
    cr = (jnp.dot(arT, bj_ref[...], preferred_element_type=jnp.float32)
          + jnp.dot(woutT_ref[...], brel_ref[...],
                    preferred_element_type=jnp.float32)
          + bout_ref[...])
    g4 = jnp.concatenate([gjT, grT], axis=0)
    c4 = jnp.concatenate([cj, cr], axis=0)
    x = x_ref[...]
    o_ref[...] = jnp.dot(g4, x, preferred_element_type=jnp.float32) + c4


def _pre_torso_body(woutT_ref, wreltjT_ref, wtT_ref, bt_ref, x_ref, o_ref):
    atjT = jnp.dot(woutT_ref[...], wreltjT_ref[...],
                   preferred_element_type=jnp.float32)
    gtT = jnp.dot(atjT, wtT_ref[...], preferred_element_type=jnp.float32)
    ct = jnp.dot(atjT, bt_ref[...], preferred_element_type=jnp.float32)
    x = x_ref[...]
    o_ref[...] = jnp.dot(gtT, x, preferred_element_type=jnp.float32) + ct


def _post_body(p0_ref, p1_ref, loc_ref, scale_ref):
    loc_ref[...] = p0_ref[0:1, :] + p0_ref[1:2, :]
    s = p1_ref[0:1, :] + p1_ref[1:2, :] + _SP_BIAS
    scale_ref[...] = jax.nn.softplus(s)


def _sc_body(yt0, yt1, yj0, yj1, stj, dtj, sjj, djj, init0, init1,
             out0, out1, src_a, src_b, dst_a, dst_b, g0_a, g0_b, g1_a, g1_b,
             stage_v, acc0, acc1, isem, gsem, ssem):
    src_v = (src_a, src_b)
    dst_v = (dst_a, dst_b)
    g0_v = (g0_a, g0_b)
    g1_v = (g1_a, g1_b)
    c = lax.axis_index("c")
    s = lax.axis_index("s")
    wid = s * NC + c
    base = s * RPT
    hb = c * NJP + base
    pltpu.sync_copy(init0.at[pl.ds(hb, RPT)], stage_v)
    pltpu.sync_copy(stage_v, acc0.at[pl.ds(base, RPT)])
    pltpu.sync_copy(init1.at[pl.ds(hb, RPT)], stage_v)
    pltpu.sync_copy(stage_v, acc1.at[pl.ds(base, RPT)])
    plsc.subcore_barrier()

    def do_edges(src_h, dst_h, t0, t1, nchunks):
        def start_idx(i, b):
            off = (wid * nchunks + i) * CH
            return (
                pltpu.async_copy(src_h.at[pl.ds(off, CH)], src_v[b], isem),
                pltpu.async_copy(dst_h.at[pl.ds(off, CH)], dst_v[b], isem),
            )

        ih = {0: start_idx(0, 0)}
        sh = {}
        for i in range(nchunks):
            b = i % 2
            for h in ih.pop(i):
                h.wait()
            gh = (pltpu.async_copy(t0.at[src_v[b]], g0_v[b], gsem),
                  pltpu.async_copy(t1.at[src_v[b]], g1_v[b], gsem))
            if i - 1 in sh:
                for h in sh.pop(i - 1):
                    h.wait()
            if i + 1 < nchunks:
                ih[i + 1] = start_idx(i + 1, 1 - b)
            for h in gh:
                h.wait()
            sh[i] = (pltpu.async_copy(g0_v[b], acc0.at[dst_v[b]],
                                      ssem, add=True),
                     pltpu.async_copy(g1_v[b], acc1.at[dst_v[b]],
                                      ssem, add=True))
        for hs in sh.values():
            for h in hs:
                h.wait()

    do_edges(stj, dtj, yt0, yt1, TJ_CH)
    do_edges(sjj, djj, yj0, yj1, JJ_CH)
    plsc.subcore_barrier()
    pltpu.sync_copy(acc0.at[pl.ds(base, RPT)], stage_v)
    pltpu.sync_copy(stage_v, out0.at[pl.ds(hb, RPT)])
    pltpu.sync_copy(acc1.at[pl.ds(base, RPT)], stage_v)
    pltpu.sync_copy(stage_v, out1.at[pl.ds(hb, RPT)])


_sc_scatter = functools.partial(
    pl.kernel,
    mesh=plsc.VectorSubcoreMesh(core_axis_name="c", subcore_axis_name="s"),
    out_type=[jax.ShapeDtypeStruct((NC * NJP,), jnp.float32),
              jax.ShapeDtypeStruct((NC * NJP,), jnp.float32)],
    scratch_types=[
        pltpu.VMEM((CH,), jnp.int32),
        pltpu.VMEM((CH,), jnp.int32),
        pltpu.VMEM((CH,), jnp.int32),
        pltpu.VMEM((CH,), jnp.int32),
        pltpu.VMEM((CH,), jnp.float32),
        pltpu.VMEM((CH,), jnp.float32),
        pltpu.VMEM((CH,), jnp.float32),
        pltpu.VMEM((CH,), jnp.float32),
        pltpu.VMEM((RPT,), jnp.float32),
        pltpu.VMEM_SHARED((NJP,), jnp.float32),
        pltpu.VMEM_SHARED((NJP,), jnp.float32),
        pltpu.SemaphoreType.DMA,
        pltpu.SemaphoreType.DMA,
        pltpu.SemaphoreType.DMA,
    ],
)(_sc_body)


def _pad_edges(ei, e_pad, n_src, n_dst):
    e = ei.shape[1]
    ar = jnp.arange(e_pad - e, dtype=jnp.int32)
    src = jnp.concatenate([ei[0].astype(jnp.int32), n_src + ar % _PAD])
    dst = jnp.concatenate([ei[1].astype(jnp.int32), n_dst + ar % _PAD])
    return src, dst


def kernel(x_joint, x_torso, edge_index_tj, edge_index_jt, edge_index_jj,
           Wj, bj, Wt, bt, Wrel_tj, brel_tj, Wroot_tj,
           Wrel_jt, brel_jt, Wroot_jt, Wrel_jj, brel_jj, Wroot_jj,
           Wout, bout):
    f32 = jnp.float32
    xjT = x_joint.T
    xtT = x_torso.T
    woutT = Wout.T
    brel_col = (brel_tj + brel_jj).reshape(64, 1)
    bj_col = bj.reshape(11, 1)
    bt_col = bt.reshape(11, 1)
    bout_col = bout.reshape(2, 1)

    blkj = 16000
    prej = pl.pallas_call(
        _pre_joint_body,
        grid=(N_J // blkj,),
        in_specs=[pl.BlockSpec((2, 64), lambda i: (0, 0)),
                  pl.BlockSpec((64, 11), lambda i: (0, 0)),
                  pl.BlockSpec((64, 11), lambda i: (0, 0)),
                  pl.BlockSpec((64, 11), lambda i: (0, 0)),
                  pl.BlockSpec((11, 2), lambda i: (0, 0)),
                  pl.BlockSpec((11, 1), lambda i: (0, 0)),
                  pl.BlockSpec((64, 1), lambda i: (0, 0)),
                  pl.BlockSpec((2, 1), lambda i: (0, 0)),
                  pl.BlockSpec((2, blkj), lambda i: (0, i))],
        out_specs=pl.BlockSpec((4, blkj), lambda i: (0, i)),
        out_shape=jax.ShapeDtypeStruct((4, N_J), f32),
    )(woutT, Wrel_jj.T, Wroot_tj.T, Wroot_jj.T, Wj.T,
      bj_col, brel_col, bout_col, xjT)

    pret = pl.pallas_call(
        _pre_torso_body,
        grid=(1,),
        in_specs=[pl.BlockSpec((2, 64), lambda i: (0, 0)),
                  pl.BlockSpec((64, 11), lambda i: (0, 0)),
                  pl.BlockSpec((11, 11), lambda i: (0, 0)),
                  pl.BlockSpec((11, 1), lambda i: (0, 0)),
                  pl.BlockSpec((11, N_T), lambda i: (0, 0))],
        out_specs=pl.BlockSpec((2, N_T), lambda i: (0, 0)),
        out_shape=jax.ShapeDtypeStruct((2, N_T), f32),
    )(woutT, Wrel_tj.T, Wt.T, bt_col, xtT)

    zpadj = jnp.zeros((_PAD,), f32)
    zrow = jnp.zeros((NJP,), f32)
    yj0 = jnp.concatenate([prej[0], zpadj])
    yj1 = jnp.concatenate([prej[1], zpadj])
    init0 = jnp.concatenate([prej[2], zpadj, zrow])
    init1 = jnp.concatenate([prej[3], zpadj, zrow])
    yt0 = jnp.concatenate([pret[0], zpadj])
    yt1 = jnp.concatenate([pret[1], zpadj])

    stj, dtj = _pad_edges(edge_index_tj, E_TJ_P, N_T, N_J)
    sjj, djj = _pad_edges(edge_index_jj, E_JJ_P, N_J, N_J)

    p0, p1 = _sc_scatter(yt0, yt1, yj0, yj1, stj, dtj, sjj, djj, init0, init1)
    p0 = p0.reshape(NC, NJP)
    p1 = p1.reshape(NC, NJP)

    blk = 16000
    loc2, scale2 = pl.pallas_call(
        _post_body,
        grid=(N_J // blk,),
        in_specs=[pl.BlockSpec((2, blk), lambda i: (0, i)),
                  pl.BlockSpec((2, blk), lambda i: (0, i))],
        out_specs=[pl.BlockSpec((1, blk), lambda i: (0, i)),
                   pl.BlockSpec((1, blk), lambda i: (0, i))],
        out_shape=[jax.ShapeDtypeStruct((1, N_J), f32),
                   jax.ShapeDtypeStruct((1, N_J), f32)],
    )(p0, p1)
    return (loc2.reshape(N_J), scale2.reshape(N_J))


# speedup vs baseline: 65.8231x; 1.8200x over previous
import functools

import numpy as np
import jax
import jax.numpy as jnp
from jax import lax
from jax.experimental import pallas as pl
from jax.experimental.pallas import tpu as pltpu
from jax.experimental.pallas import tpu_sc as plsc

N_J = 80000
N_T = 20000
_PAD = 2048
_PADT = 2528
NJP = N_J + _PAD
NTP = N_T + _PADT
NC = 2
NS = 16
TPT = NTP // NS
NW = NC * NS
CH = 10240
TJ_CH = 2
JJ_CH = 6
E_TJ_P = NW * TJ_CH * CH
E_JJ_P = NW * JJ_CH * CH
RPT = NJP // NS
_SP_BIAS = float(np.log(np.exp(1.0) - 1.0))


def _pre_joint_body(woutT_ref, wreljjT_ref, wrtjT_ref, wrjjT_ref, wjT_ref,
                    bj_ref, brel_ref, bout_ref, x_ref, o_ref):
    ajjT = jnp.dot(woutT_ref[...], wreljjT_ref[...],
                   preferred_element_type=jnp.float32)
    arT = jnp.dot(woutT_ref[...], wrtjT_ref[...] + wrjjT_ref[...],
                  preferred_element_type=jnp.float32)
    gjT = jnp.dot(ajjT, wjT_ref[...], preferred_element_type=jnp.float32)
    grT = jnp.dot(arT, wjT_ref[...], preferred_element_type=jnp.float32)
    cj = jnp.dot(ajjT, bj_ref[...], preferred_element_type=jnp.float32)
    cr = (jnp.dot(arT, bj_ref[...], preferred_element_type=jnp.float32)
          + jnp.dot(woutT_ref[...], brel_ref[...],
                    preferred_element_type=jnp.float32)
          + bout_ref[...])
    g4 = jnp.concatenate([gjT, grT], axis=0)
    c4 = jnp.concatenate([cj, cr], axis=0)
    x = x_ref[...]
    o_ref[...] = jnp.dot(g4, x, preferred_element_type=jnp.float32) + c4


def _pre_torso_body(woutT_ref, wreltjT_ref, wtT_ref, bt_ref, x_ref, o_ref):
    atjT = jnp.dot(woutT_ref[...], wreltjT_ref[...],
                   preferred_element_type=jnp.float32)
    gtT = jnp.dot(atjT, wtT_ref[...], preferred_element_type=jnp.float32)
    ct = jnp.dot(atjT, bt_ref[...], preferred_element_type=jnp.float32)
    x = x_ref[...]
    o_ref[...] = jnp.dot(gtT, x, preferred_element_type=jnp.float32) + ct


def _post_body(p0_ref, p1_ref, loc_ref, scale_ref):
    loc_ref[...] = p0_ref[0:1, :] + p0_ref[1:2, :]
    s = p1_ref[0:1, :] + p1_ref[1:2, :] + _SP_BIAS
    scale_ref[...] = jax.nn.softplus(s)


def _sc_body(yt0, yt1, yj0, yj1, stj, dtj, sjj, djj, init0, init1,
             out0, out1, src_a, src_b, dst_a, dst_b, g0_a, g0_b, g1_a, g1_b,
             stage_v, acc0, acc1, ts0, ts1, js0, js1, isem, gsem, ssem):
    src_v = (src_a, src_b)
    dst_v = (dst_a, dst_b)
    g0_v = (g0_a, g0_b)
    g1_v = (g1_a, g1_b)
    c = lax.axis_index("c")
    s = lax.axis_index("s")
    wid = s * NC + c
    base = s * RPT
    hb = c * NJP + base
    tb = s * TPT
    for src_t, dst_t, src_off, dst_off, n in (
            (init0, acc0, hb, base, RPT), (init1, acc1, hb, base, RPT),
            (yj0, js0, base, base, RPT), (yj1, js1, base, base, RPT),
            (yt0, ts0, tb, tb, TPT), (yt1, ts1, tb, tb, TPT)):
        pltpu.sync_copy(src_t.at[pl.ds(src_off, n)], stage_v.at[pl.ds(0, n)])
        pltpu.sync_copy(stage_v.at[pl.ds(0, n)], dst_t.at[pl.ds(dst_off, n)])
    plsc.subcore_barrier()

    def do_edges(src_h, dst_h, t0, t1, nchunks):
        def start_idx(i, b):
            off = (wid * nchunks + i) * CH
            return (
                pltpu.async_copy(src_h.at[pl.ds(off, CH)], src_v[b], isem),
                pltpu.async_copy(dst_h.at[pl.ds(off, CH)], dst_v[b], isem),
            )

        ih = {0: start_idx(0, 0)}
        sh = {}
        for i in range(nchunks):
            b = i % 2
            for h in ih.pop(i):
                h.wait()
            gh = (pltpu.async_copy(t0.at[src_v[b]], g0_v[b], gsem),
                  pltpu.async_copy(t1.at[src_v[b]], g1_v[b], gsem))
            if i - 1 in sh:
                for h in sh.pop(i - 1):
                    h.wait()
            if i + 1 < nchunks:
                ih[i + 1] = start_idx(i + 1, 1 - b)
            for h in gh:
                h.wait()
            sh[i] = (pltpu.async_copy(g0_v[b], acc0.at[dst_v[b]],
                                      ssem, add=True),
                     pltpu.async_copy(g1_v[b], acc1.at[dst_v[b]],
                                      ssem, add=True))
        for hs in sh.values():
            for h in hs:
                h.wait()

    do_edges(stj, dtj, ts0, ts1, TJ_CH)
    do_edges(sjj, djj, js0, js1, JJ_CH)
    plsc.subcore_barrier()
    pltpu.sync_copy(acc0.at[pl.ds(base, RPT)], stage_v)
    pltpu.sync_copy(stage_v, out0.at[pl.ds(hb, RPT)])
    pltpu.sync_copy(acc1.at[pl.ds(base, RPT)], stage_v)
    pltpu.sync_copy(stage_v, out1.at[pl.ds(hb, RPT)])


_sc_scatter = functools.partial(
    pl.kernel,
    mesh=plsc.VectorSubcoreMesh(core_axis_name="c", subcore_axis_name="s"),
    out_type=[jax.ShapeDtypeStruct((NC * NJP,), jnp.float32),
              jax.ShapeDtypeStruct((NC * NJP,), jnp.float32)],
    scratch_types=[
        pltpu.VMEM((CH,), jnp.int32),
        pltpu.VMEM((CH,), jnp.int32),
        pltpu.VMEM((CH,), jnp.int32),
        pltpu.VMEM((CH,), jnp.int32),
        pltpu.VMEM((CH,), jnp.float32),
        pltpu.VMEM((CH,), jnp.float32),
        pltpu.VMEM((CH,), jnp.float32),
        pltpu.VMEM((CH,), jnp.float32),
        pltpu.VMEM((RPT,), jnp.float32),
        pltpu.VMEM_SHARED((NJP,), jnp.float32),
        pltpu.VMEM_SHARED((NJP,), jnp.float32),
        pltpu.VMEM_SHARED((NTP,), jnp.float32),
        pltpu.VMEM_SHARED((NTP,), jnp.float32),
        pltpu.VMEM_SHARED((NJP,), jnp.float32),
        pltpu.VMEM_SHARED((NJP,), jnp.float32),
        pltpu.SemaphoreType.DMA,
        pltpu.SemaphoreType.DMA,
        pltpu.SemaphoreType.DMA,
    ],
)(_sc_body)


def _pad_edges(ei, e_pad, n_src, src_mod, n_dst):
    e = ei.shape[1]
    ar = jnp.arange(e_pad - e, dtype=jnp.int32)
    src = jnp.concatenate([ei[0].astype(jnp.int32), n_src + ar % src_mod])
    dst = jnp.concatenate([ei[1].astype(jnp.int32), n_dst + ar % _PAD])
    return src, dst


def kernel(x_joint, x_torso, edge_index_tj, edge_index_jt, edge_index_jj,
           Wj, bj, Wt, bt, Wrel_tj, brel_tj, Wroot_tj,
           Wrel_jt, brel_jt, Wroot_jt, Wrel_jj, brel_jj, Wroot_jj,
           Wout, bout):
    f32 = jnp.float32
    xjT = x_joint.T
    xtT = x_torso.T
    woutT = Wout.T
    brel_col = (brel_tj + brel_jj).reshape(64, 1)
    bj_col = bj.reshape(11, 1)
    bt_col = bt.reshape(11, 1)
    bout_col = bout.reshape(2, 1)

    blkj = 16000
    prej = pl.pallas_call(
        _pre_joint_body,
        grid=(N_J // blkj,),
        in_specs=[pl.BlockSpec((2, 64), lambda i: (0, 0)),
                  pl.BlockSpec((64, 11), lambda i: (0, 0)),
                  pl.BlockSpec((64, 11), lambda i: (0, 0)),
                  pl.BlockSpec((64, 11), lambda i: (0, 0)),
                  pl.BlockSpec((11, 2), lambda i: (0, 0)),
                  pl.BlockSpec((11, 1), lambda i: (0, 0)),
                  pl.BlockSpec((64, 1), lambda i: (0, 0)),
                  pl.BlockSpec((2, 1), lambda i: (0, 0)),
                  pl.BlockSpec((2, blkj), lambda i: (0, i))],
        out_specs=pl.BlockSpec((4, blkj), lambda i: (0, i)),
        out_shape=jax.ShapeDtypeStruct((4, N_J), f32),
    )(woutT, Wrel_jj.T, Wroot_tj.T, Wroot_jj.T, Wj.T,
      bj_col, brel_col, bout_col, xjT)

    pret = pl.pallas_call(
        _pre_torso_body,
        grid=(1,),
        in_specs=[pl.BlockSpec((2, 64), lambda i: (0, 0)),
                  pl.BlockSpec((64, 11), lambda i: (0, 0)),
                  pl.BlockSpec((11, 11), lambda i: (0, 0)),
                  pl.BlockSpec((11, 1), lambda i: (0, 0)),
                  pl.BlockSpec((11, N_T), lambda i: (0, 0))],
        out_specs=pl.BlockSpec((2, N_T), lambda i: (0, 0)),
        out_shape=jax.ShapeDtypeStruct((2, N_T), f32),
    )(woutT, Wrel_tj.T, Wt.T, bt_col, xtT)

    zpadj = jnp.zeros((_PAD,), f32)
    zpadt = jnp.zeros((_PADT,), f32)
    zrow = jnp.zeros((NJP,), f32)
    yj0 = jnp.concatenate([prej[0], zpadj])
    yj1 = jnp.concatenate([prej[1], zpadj])
    init0 = jnp.concatenate([prej[2], zpadj, zrow])
    init1 = jnp.concatenate([prej[3], zpadj, zrow])
    yt0 = jnp.concatenate([pret[0], zpadt])
    yt1 = jnp.concatenate([pret[1], zpadt])

    stj, dtj = _pad_edges(edge_index_tj, E_TJ_P, N_T, _PADT, N_J)
    sjj, djj = _pad_edges(edge_index_jj, E_JJ_P, N_J, _PAD, N_J)

    p0, p1 = _sc_scatter(yt0, yt1, yj0, yj1, stj, dtj, sjj, djj, init0, init1)
    p0 = p0.reshape(NC, NJP)
    p1 = p1.reshape(NC, NJP)

    blk = 16000
    loc2, scale2 = pl.pallas_call(
        _post_body,
        grid=(N_J // blk,),
        in_specs=[pl.BlockSpec((2, blk), lambda i: (0, i)),
                  pl.BlockSpec((2, blk), lambda i: (0, i))],
        out_specs=[pl.BlockSpec((1, blk), lambda i: (0, i)),
                   pl.BlockSpec((1, blk), lambda i: (0, i))],
        out_shape=[jax.ShapeDtypeStruct((1, N_J), f32),
                   jax.ShapeDtypeStruct((1, N_J), f32)],
    )(p0, p1)
    return (loc2.reshape(N_J), scale2.reshape(N_J))
